# baseline (speedup 1.0000x reference)
import jax
import jax.numpy as jnp
from jax.experimental import pallas as pl
from jax.experimental.pallas import tpu as pltpu

B = 32
SEQ = 3000
ENC = 256
NC = 5
RW = 256
NE = 4

K = 600
NK = SEQ // K


def _moe_body(eegT, eogT, w0, w1, w2a, w2b, w3a, w3b, wr1a, wr1b,
              wh0, wh1, wh2, wh3, b1, wr2, b2,
              out_logits, out_rw, out_eo, out_il,
              a0, a1, a2a, a2b, a3a, a3b, ar):
    k = pl.program_id(0)
    x = eegT[0]
    y = eogT[0]

    @pl.when(k == 0)
    def _init():
        z = jnp.zeros((B, ENC), jnp.float32)
        a0[...] = z
        a1[...] = z
        a2a[...] = z
        a2b[...] = z
        a3a[...] = z
        a3b[...] = z
        ar[...] = z

    def dot(a, b):
        return jnp.dot(a, b, preferred_element_type=jnp.float32)

    a0[...] += dot(x, w0[...])
    a1[...] += dot(y, w1[...])
    a2a[...] += dot(x, w2a[...])
    a2b[...] += dot(y, w2b[...])
    a3a[...] += dot(x, w3a[...])
    a3b[...] += dot(y, w3b[...])
    ar[...] += dot(x, wr1a[...]) + dot(y, wr1b[...])

    @pl.when(k == NK - 1)
    def _finalize():
        relu = lambda t: jnp.maximum(t, 0.0)
        h0 = relu(a0[...])
        h1 = relu(a1[...])
        h2a = relu(a2a[...])
        h2b = relu(a2b[...])
        h3a = relu(a3a[...])
        h3b = relu(a3b[...])
        eo0 = dot(h0, wh0[...])
        eo1 = dot(h1, wh1[...])
        eo2 = dot(h2a, wh2[0:ENC, :]) + dot(h2b, wh2[ENC:2 * ENC, :])
        eo3 = dot(h3a, wh3[0:ENC, :]) + dot(h3b, wh3[ENC:2 * ENC, :])
        hr = relu(ar[...] + b1[...])
        rl = dot(hr, wr2[...]) + b2[...]
        m = jnp.max(rl, axis=-1, keepdims=True)
        ex = jnp.exp(rl - m)
        rw = ex / jnp.sum(ex, axis=-1, keepdims=True)
        out_rw[...] = rw
        out_logits[...] = (rw[:, 0:1] * eo0 + rw[:, 1:2] * eo1
                           + rw[:, 2:3] * eo2 + rw[:, 3:4] * eo3)
        out_eo[0] = eo0
        out_eo[1] = eo1
        out_eo[2] = eo2
        out_eo[3] = eo3
        avg = 0.25 * (eo0 + eo1 + eo2 + eo3)
        inv = 1.0 / (B * NC)
        out_il[0:1, 0:1] = (jnp.sum((eo0 - avg) ** 2, keepdims=True) * inv)
        out_il[0:1, 1:2] = (jnp.sum((eo1 - avg) ** 2, keepdims=True) * inv)
        out_il[0:1, 2:3] = (jnp.sum((eo2 - avg) ** 2, keepdims=True) * inv)
        out_il[0:1, 3:4] = (jnp.sum((eo3 - avg) ** 2, keepdims=True) * inv)


def kernel(eeg, eog, We_eeg0, We_eog0, Wh0, We_eeg1, We_eog1, Wh1,
           We_eeg2, We_eog2, Wh2, We_eeg3, We_eog3, Wh3,
           Wr1, br1, Wr2, br2):
    eegT = jnp.transpose(eeg.reshape(B, NK, K), (1, 0, 2))
    eogT = jnp.transpose(eog.reshape(B, NK, K), (1, 0, 2))
    b1 = br1.reshape(1, RW)
    b2 = br2.reshape(1, NE)

    xspec = pl.BlockSpec((1, B, K), lambda k: (k, 0, 0))
    wspec = pl.BlockSpec((K, ENC), lambda k: (k, 0))
    wr1a_spec = pl.BlockSpec((K, ENC), lambda k: (k, 0))
    wr1b_spec = pl.BlockSpec((K, ENC), lambda k: (k + NK, 0))

    def full(shape):
        return pl.BlockSpec(shape, lambda k: (0,) * len(shape))

    out_shape = (
        jax.ShapeDtypeStruct((B, NC), jnp.float32),
        jax.ShapeDtypeStruct((B, NE), jnp.float32),
        jax.ShapeDtypeStruct((NE, B, NC), jnp.float32),
        jax.ShapeDtypeStruct((1, NE), jnp.float32),
    )
    out_specs = (full((B, NC)), full((B, NE)), full((NE, B, NC)),
                 full((1, NE)))

    logits, rw, eo, il = pl.pallas_call(
        _moe_body,
        grid=(NK,),
        in_specs=[xspec, xspec,
                  wspec, wspec, wspec, wspec, wspec, wspec,
                  wr1a_spec, wr1b_spec,
                  full((ENC, NC)), full((ENC, NC)),
                  full((2 * ENC, NC)), full((2 * ENC, NC)),
                  full((1, RW)), full((RW, NE)), full((1, NE))],
        out_specs=out_specs,
        out_shape=out_shape,
        scratch_shapes=[pltpu.VMEM((B, ENC), jnp.float32)] * 7,
        compiler_params=pltpu.CompilerParams(
            dimension_semantics=("arbitrary",)),
    )(eegT, eogT, We_eeg0, We_eog1, We_eeg2, We_eog2, We_eeg3, We_eog3,
      Wr1, Wr1, Wh0, Wh1, Wh2, Wh3, b1, Wr2, b2)
    return logits, rw, eo, il.reshape(NE)


# speedup vs baseline: 1.0090x; 1.0090x over previous
import jax
import jax.numpy as jnp
from jax import lax
from jax.experimental import pallas as pl
from jax.experimental.pallas import tpu as pltpu

B = 32
SEQ = 3000
ENC = 256
NC = 5
RW = 256
NE = 4

K = 512
NK = (SEQ + K - 1) // K
TAIL = SEQ - (NK - 1) * K


def _moe_body(eeg, eog, w0, w1, w2a, w2b, w3a, w3b, wr1a, wr1b,
              wh0, wh1, wh2, wh3, b1, wr2, b2,
              out_logits, out_rw, out_eo, out_il,
              a0, a1, a2a, a2b, a3a, a3b, ar):
    k = pl.program_id(0)

    @pl.when(k == 0)
    def _init():
        z = jnp.zeros((B, ENC), jnp.float32)
        a0[...] = z
        a1[...] = z
        a2a[...] = z
        a2b[...] = z
        a3a[...] = z
        a3b[...] = z
        ar[...] = z

    def dot(a, b):
        return jnp.dot(a, b, preferred_element_type=jnp.float32)

    def accumulate(x, y, wmask):
        a0[...] += dot(x, wmask(w0[...]))
        a1[...] += dot(y, wmask(w1[...]))
        a2a[...] += dot(x, wmask(w2a[...]))
        a2b[...] += dot(y, wmask(w2b[...]))
        a3a[...] += dot(x, wmask(w3a[...]))
        a3b[...] += dot(y, wmask(w3b[...]))
        ar[...] += dot(x, wmask(wr1a[0])) + dot(y, wmask(wr1b[0]))

    @pl.when(k < NK - 1)
    def _interior():
        accumulate(eeg[...], eog[...], lambda w: w)

    @pl.when(k == NK - 1)
    def _final_step():
        colmask = lax.broadcasted_iota(jnp.int32, (B, K), 1) < TAIL
        rowmask = lax.broadcasted_iota(jnp.int32, (K, ENC), 0) < TAIL
        x = jnp.where(colmask, eeg[...], 0.0)
        y = jnp.where(colmask, eog[...], 0.0)
        zf = jnp.zeros((K, ENC), jnp.float32)
        accumulate(x, y, lambda w: jnp.where(rowmask, w, zf))

        relu = lambda t: jnp.maximum(t, 0.0)
        h0 = relu(a0[...])
        h1 = relu(a1[...])
        h2a = relu(a2a[...])
        h2b = relu(a2b[...])
        h3a = relu(a3a[...])
        h3b = relu(a3b[...])
        eo0 = dot(h0, wh0[...])
        eo1 = dot(h1, wh1[...])
        eo2 = dot(h2a, wh2[0:ENC, :]) + dot(h2b, wh2[ENC:2 * ENC, :])
        eo3 = dot(h3a, wh3[0:ENC, :]) + dot(h3b, wh3[ENC:2 * ENC, :])
        hr = relu(ar[...] + b1[...])
        rl = dot(hr, wr2[...]) + b2[...]
        m = jnp.max(rl, axis=-1, keepdims=True)
        ex = jnp.exp(rl - m)
        rw = ex / jnp.sum(ex, axis=-1, keepdims=True)
        out_rw[...] = rw
        out_logits[...] = (rw[:, 0:1] * eo0 + rw[:, 1:2] * eo1
                           + rw[:, 2:3] * eo2 + rw[:, 3:4] * eo3)
        out_eo[0] = eo0
        out_eo[1] = eo1
        out_eo[2] = eo2
        out_eo[3] = eo3
        avg = 0.25 * (eo0 + eo1 + eo2 + eo3)
        inv = 1.0 / (B * NC)
        out_il[0:1, 0:1] = (jnp.sum((eo0 - avg) ** 2, keepdims=True) * inv)
        out_il[0:1, 1:2] = (jnp.sum((eo1 - avg) ** 2, keepdims=True) * inv)
        out_il[0:1, 2:3] = (jnp.sum((eo2 - avg) ** 2, keepdims=True) * inv)
        out_il[0:1, 3:4] = (jnp.sum((eo3 - avg) ** 2, keepdims=True) * inv)


def kernel(eeg, eog, We_eeg0, We_eog0, Wh0, We_eeg1, We_eog1, Wh1,
           We_eeg2, We_eog2, Wh2, We_eeg3, We_eog3, Wh3,
           Wr1, br1, Wr2, br2):
    Wr1h = Wr1.reshape(2, SEQ, RW)
    b1 = br1.reshape(1, RW)
    b2 = br2.reshape(1, NE)

    xspec = pl.BlockSpec((B, K), lambda k: (0, k))
    wspec = pl.BlockSpec((K, ENC), lambda k: (k, 0))
    wr1a_spec = pl.BlockSpec((1, K, ENC), lambda k: (0, k, 0))
    wr1b_spec = pl.BlockSpec((1, K, ENC), lambda k: (1, k, 0))

    def full(shape):
        return pl.BlockSpec(shape, lambda k: (0,) * len(shape))

    out_shape = (
        jax.ShapeDtypeStruct((B, NC), jnp.float32),
        jax.ShapeDtypeStruct((B, NE), jnp.float32),
        jax.ShapeDtypeStruct((NE, B, NC), jnp.float32),
        jax.ShapeDtypeStruct((1, NE), jnp.float32),
    )
    out_specs = (full((B, NC)), full((B, NE)), full((NE, B, NC)),
                 full((1, NE)))

    logits, rw, eo, il = pl.pallas_call(
        _moe_body,
        grid=(NK,),
        in_specs=[xspec, xspec,
                  wspec, wspec, wspec, wspec, wspec, wspec,
                  wr1a_spec, wr1b_spec,
                  full((ENC, NC)), full((ENC, NC)),
                  full((2 * ENC, NC)), full((2 * ENC, NC)),
                  full((1, RW)), full((RW, NE)), full((1, NE))],
        out_specs=out_specs,
        out_shape=out_shape,
        scratch_shapes=[pltpu.VMEM((B, ENC), jnp.float32)] * 7,
        compiler_params=pltpu.CompilerParams(
            dimension_semantics=("arbitrary",)),
    )(eeg, eog, We_eeg0, We_eog1, We_eeg2, We_eog2, We_eeg3, We_eog3,
      Wr1h, Wr1h, Wh0, Wh1, Wh2, Wh3, b1, Wr2, b2)
    return logits, rw, eo, il.reshape(NE)


# speedup vs baseline: 1.0312x; 1.0220x over previous
import jax
import jax.numpy as jnp
from jax import lax
from jax.experimental import pallas as pl
from jax.experimental.pallas import tpu as pltpu

B = 32
SEQ = 3000
ENC = 256
NC = 5
RW = 256
NE = 4

K = 1024
NK = (SEQ + K - 1) // K
TAIL = SEQ - (NK - 1) * K


def _moe_body(eeg, eog, w0, w1, w2a, w2b, w3a, w3b, wr1a, wr1b,
              wh0, wh1, wh2, wh3, b1, wr2, b2,
              out_logits, out_rw, out_eo, out_il,
              a0, a1, a2a, a2b, a3a, a3b, ar):
    k = pl.program_id(0)

    @pl.when(k == 0)
    def _init():
        z = jnp.zeros((B, ENC), jnp.float32)
        a0[...] = z
        a1[...] = z
        a2a[...] = z
        a2b[...] = z
        a3a[...] = z
        a3b[...] = z
        ar[...] = z

    def dot(a, b):
        return jnp.dot(a, b, preferred_element_type=jnp.float32)

    def accumulate(x, y, wmask):
        a0[...] += dot(x, wmask(w0[...]))
        a1[...] += dot(y, wmask(w1[...]))
        a2a[...] += dot(x, wmask(w2a[...]))
        a2b[...] += dot(y, wmask(w2b[...]))
        a3a[...] += dot(x, wmask(w3a[...]))
        a3b[...] += dot(y, wmask(w3b[...]))
        ar[...] += dot(x, wmask(wr1a[0])) + dot(y, wmask(wr1b[0]))

    @pl.when(k < NK - 1)
    def _interior():
        accumulate(eeg[...], eog[...], lambda w: w)

    @pl.when(k == NK - 1)
    def _final_step():
        colmask = lax.broadcasted_iota(jnp.int32, (B, K), 1) < TAIL
        rowmask = lax.broadcasted_iota(jnp.int32, (K, ENC), 0) < TAIL
        x = jnp.where(colmask, eeg[...], 0.0)
        y = jnp.where(colmask, eog[...], 0.0)
        zf = jnp.zeros((K, ENC), jnp.float32)
        accumulate(x, y, lambda w: jnp.where(rowmask, w, zf))

        relu = lambda t: jnp.maximum(t, 0.0)
        h0 = relu(a0[...])
        h1 = relu(a1[...])
        h2a = relu(a2a[...])
        h2b = relu(a2b[...])
        h3a = relu(a3a[...])
        h3b = relu(a3b[...])
        eo0 = dot(h0, wh0[...])
        eo1 = dot(h1, wh1[...])
        eo2 = dot(h2a, wh2[0:ENC, :]) + dot(h2b, wh2[ENC:2 * ENC, :])
        eo3 = dot(h3a, wh3[0:ENC, :]) + dot(h3b, wh3[ENC:2 * ENC, :])
        hr = relu(ar[...] + b1[...])
        rl = dot(hr, wr2[...]) + b2[...]
        m = jnp.max(rl, axis=-1, keepdims=True)
        ex = jnp.exp(rl - m)
        rw = ex / jnp.sum(ex, axis=-1, keepdims=True)
        out_rw[...] = rw
        out_logits[...] = (rw[:, 0:1] * eo0 + rw[:, 1:2] * eo1
                           + rw[:, 2:3] * eo2 + rw[:, 3:4] * eo3)
        out_eo[0] = eo0
        out_eo[1] = eo1
        out_eo[2] = eo2
        out_eo[3] = eo3
        avg = 0.25 * (eo0 + eo1 + eo2 + eo3)
        inv = 1.0 / (B * NC)
        out_il[0:1, 0:1] = (jnp.sum((eo0 - avg) ** 2, keepdims=True) * inv)
        out_il[0:1, 1:2] = (jnp.sum((eo1 - avg) ** 2, keepdims=True) * inv)
        out_il[0:1, 2:3] = (jnp.sum((eo2 - avg) ** 2, keepdims=True) * inv)
        out_il[0:1, 3:4] = (jnp.sum((eo3 - avg) ** 2, keepdims=True) * inv)


def kernel(eeg, eog, We_eeg0, We_eog0, Wh0, We_eeg1, We_eog1, Wh1,
           We_eeg2, We_eog2, Wh2, We_eeg3, We_eog3, Wh3,
           Wr1, br1, Wr2, br2):
    Wr1h = Wr1.reshape(2, SEQ, RW)
    b1 = br1.reshape(1, RW)
    b2 = br2.reshape(1, NE)

    xspec = pl.BlockSpec((B, K), lambda k: (0, k))
    wspec = pl.BlockSpec((K, ENC), lambda k: (k, 0))
    wr1a_spec = pl.BlockSpec((1, K, ENC), lambda k: (0, k, 0))
    wr1b_spec = pl.BlockSpec((1, K, ENC), lambda k: (1, k, 0))

    def full(shape):
        return pl.BlockSpec(shape, lambda k: (0,) * len(shape))

    out_shape = (
        jax.ShapeDtypeStruct((B, NC), jnp.float32),
        jax.ShapeDtypeStruct((B, NE), jnp.float32),
        jax.ShapeDtypeStruct((NE, B, NC), jnp.float32),
        jax.ShapeDtypeStruct((1, NE), jnp.float32),
    )
    out_specs = (full((B, NC)), full((B, NE)), full((NE, B, NC)),
                 full((1, NE)))

    logits, rw, eo, il = pl.pallas_call(
        _moe_body,
        grid=(NK,),
        in_specs=[xspec, xspec,
                  wspec, wspec, wspec, wspec, wspec, wspec,
                  wr1a_spec, wr1b_spec,
                  full((ENC, NC)), full((ENC, NC)),
                  full((2 * ENC, NC)), full((2 * ENC, NC)),
                  full((1, RW)), full((RW, NE)), full((1, NE))],
        out_specs=out_specs,
        out_shape=out_shape,
        scratch_shapes=[pltpu.VMEM((B, ENC), jnp.float32)] * 7,
        compiler_params=pltpu.CompilerParams(
            dimension_semantics=("arbitrary",)),
    )(eeg, eog, We_eeg0, We_eog1, We_eeg2, We_eog2, We_eeg3, We_eog3,
      Wr1h, Wr1h, Wh0, Wh1, Wh2, Wh3, b1, Wr2, b2)
    return logits, rw, eo, il.reshape(NE)
